# baseline (speedup 1.0000x reference)
import functools

import jax
import jax.numpy as jnp
from jax.experimental import pallas as pl
from jax.experimental.pallas import tpu as pltpu

_S = 512


def _body(xb_ref, halo_ref, cls_ref, pos_ref, out_ref, *, s):
    k = pl.program_id(0)
    first = jnp.where(k == 0, cls_ref[0], halo_ref[0, 0])
    xblk = xb_ref[0]
    rolled = pltpu.roll(xblk, shift=1, axis=0)
    row_ids = jax.lax.broadcasted_iota(jnp.int32, xblk.shape, 0)
    shifted = jnp.where(row_ids == 0, first, rolled)
    out_ref[0] = shifted + pos_ref[...]


def kernel(x, cls_token, pos_table):
    batch, seq_len, d = x.shape
    s = _S
    kx = seq_len // s
    grid_k = kx + 1
    halo = x[:, s - 1 :: s, :].reshape(batch, kx, 1, d)

    def xb_index(k, b):
        return (jnp.where(k == kx, batch - 1, b), jnp.minimum(k, kx - 1), 0)

    def halo_index(k, b):
        return (b, jnp.maximum(k - 1, 0), 0, 0)

    out = pl.pallas_call(
        functools.partial(_body, s=s),
        grid=(grid_k, batch),
        in_specs=[
            pl.BlockSpec((1, s, d), xb_index),
            pl.BlockSpec((1, 1, 1, d), halo_index),
            pl.BlockSpec((1, 1, d), lambda k, b: (0, 0, 0)),
            pl.BlockSpec((s, d), lambda k, b: (k, 0)),
        ],
        out_specs=pl.BlockSpec((1, s, d), lambda k, b: (b, k, 0)),
        out_shape=jax.ShapeDtypeStruct((batch, seq_len + 1, d), x.dtype),
        compiler_params=pltpu.CompilerParams(
            dimension_semantics=("arbitrary", "parallel"),
        ),
    )(x, halo, cls_token, pos_table)
    return out


# speedup vs baseline: 1.2093x; 1.2093x over previous
import functools

import jax
import jax.numpy as jnp
from jax.experimental import pallas as pl
from jax.experimental.pallas import tpu as pltpu

_S = 2048


def _body(xb_ref, halo_ref, cls_ref, pos_ref, out_ref, *, s):
    k = pl.program_id(0)
    first = jnp.where(k == 0, cls_ref[0], halo_ref[0, 0, 7:8, :])
    xblk = xb_ref[0, 0]
    rolled = pltpu.roll(xblk, shift=1, axis=0)
    row_ids = jax.lax.broadcasted_iota(jnp.int32, xblk.shape, 0)
    shifted = jnp.where(row_ids == 0, first, rolled)
    out_ref[0] = shifted + pos_ref[...]


def kernel(x, cls_token, pos_table):
    batch, seq_len, d = x.shape
    s = _S
    kx = seq_len // s
    grid_k = kx + 1
    x4 = x.reshape(batch, kx, s, d)

    def xb_index(k, b):
        return (jnp.where(k == kx, batch - 1, b), jnp.minimum(k, kx - 1), 0, 0)

    def halo_index(k, b):
        return (b, jnp.maximum(k - 1, 0), s // 8 - 1, 0)

    out = pl.pallas_call(
        functools.partial(_body, s=s),
        grid=(grid_k, batch),
        in_specs=[
            pl.BlockSpec((1, 1, s, d), xb_index),
            pl.BlockSpec((1, 1, 8, d), halo_index),
            pl.BlockSpec((1, 1, d), lambda k, b: (0, 0, 0)),
            pl.BlockSpec((s, d), lambda k, b: (k, 0)),
        ],
        out_specs=pl.BlockSpec((1, s, d), lambda k, b: (b, k, 0)),
        out_shape=jax.ShapeDtypeStruct((batch, seq_len + 1, d), x.dtype),
        compiler_params=pltpu.CompilerParams(
            dimension_semantics=("arbitrary", "parallel"),
        ),
    )(x4, x4, cls_token, pos_table)
    return out


# speedup vs baseline: 1.2107x; 1.0011x over previous
import functools

import jax
import jax.numpy as jnp
from jax.experimental import pallas as pl
from jax.experimental.pallas import tpu as pltpu

_S = 2048


def _body(xb_ref, halo_ref, cls_ref, pos_ref, out_ref, *, s):
    k = pl.program_id(0)
    first = jnp.where(k == 0, cls_ref[0], halo_ref[0, 0, 7:8, :])
    xblk = xb_ref[0, 0]
    rolled = pltpu.roll(xblk, shift=1, axis=0)
    out_ref[0] = rolled + pos_ref[...]
    out_ref[0, 0:1, :] = first + pos_ref[0:1, :]


def kernel(x, cls_token, pos_table):
    batch, seq_len, d = x.shape
    s = _S
    kx = seq_len // s
    grid_k = kx + 1
    x4 = x.reshape(batch, kx, s, d)

    def xb_index(k, b):
        return (jnp.where(k == kx, batch - 1, b), jnp.minimum(k, kx - 1), 0, 0)

    def halo_index(k, b):
        return (b, jnp.maximum(k - 1, 0), s // 8 - 1, 0)

    out = pl.pallas_call(
        functools.partial(_body, s=s),
        grid=(grid_k, batch),
        in_specs=[
            pl.BlockSpec((1, 1, s, d), xb_index),
            pl.BlockSpec((1, 1, 8, d), halo_index),
            pl.BlockSpec((1, 1, d), lambda k, b: (0, 0, 0)),
            pl.BlockSpec((s, d), lambda k, b: (k, 0)),
        ],
        out_specs=pl.BlockSpec((1, s, d), lambda k, b: (b, k, 0)),
        out_shape=jax.ShapeDtypeStruct((batch, seq_len + 1, d), x.dtype),
        compiler_params=pltpu.CompilerParams(
            dimension_semantics=("arbitrary", "parallel"),
        ),
    )(x4, x4, cls_token, pos_table)
    return out


# speedup vs baseline: 1.2313x; 1.0170x over previous
import functools

import jax
import jax.numpy as jnp
from jax.experimental import pallas as pl
from jax.experimental.pallas import tpu as pltpu

_S = 1024
_NBUF = 4


def _body(x_ref, cls_ref, pos_ref, out_ref,
          in_bufs, out_bufs, pos_vmem, halo, tail_buf,
          in_sems, out_sems, pos_sems, tail_sem,
          *, batch, seq_len, d, s, nbuf):
    kx = seq_len // s
    steps = kx * batch

    def in_dma(step):
        k = step // batch
        b = step % batch
        slot = jax.lax.rem(step, nbuf)
        return pltpu.make_async_copy(
            x_ref.at[b, pl.ds(k * s, s), :],
            in_bufs.at[slot],
            in_sems.at[slot],
        )

    def out_dma(step):
        k = step // batch
        b = step % batch
        slot = jax.lax.rem(step, nbuf)
        return pltpu.make_async_copy(
            out_bufs.at[slot],
            out_ref.at[b, pl.ds(k * s, s), :],
            out_sems.at[slot],
        )

    def pos_dma(k):
        return pltpu.make_async_copy(
            pos_ref.at[pl.ds(k * s, s), :],
            pos_vmem.at[pl.ds(k * s, s), :],
            pos_sems.at[k],
        )

    for k in range(kx):
        pos_dma(k).start()
    pltpu.make_async_copy(
        pos_ref.at[pl.ds(kx * s, 1), :], tail_buf, tail_sem
    ).start()
    for i in range(nbuf - 1):
        in_dma(i).start()

    def step_fn(step, _):
        k = step // batch
        b = step % batch
        slot = jax.lax.rem(step, nbuf)

        @pl.when(step + nbuf - 1 < steps)
        def _():
            in_dma(step + nbuf - 1).start()

        @pl.when(b == 0)
        def _():
            pos_dma(k).wait()

        @pl.when(step >= nbuf)
        def _():
            out_dma(step - nbuf).wait()

        in_dma(step).wait()
        xblk = in_bufs[slot]
        first = jnp.where(k == 0, cls_ref[0], halo[b, 0:1, :])
        rolled = pltpu.roll(xblk, shift=1, axis=0)
        out_bufs[slot] = rolled + pos_vmem[pl.ds(k * s, s), :]
        out_bufs[slot, 0:1, :] = first + pos_vmem[pl.ds(k * s, 1), :]
        halo[b, 0:1, :] = xblk[s - 1 : s, :]
        out_dma(step).start()
        return ()

    jax.lax.fori_loop(0, steps, step_fn, (), unroll=False)

    pltpu.make_async_copy(
        pos_ref.at[pl.ds(kx * s, 1), :], tail_buf, tail_sem
    ).wait()
    for b in range(batch):
        halo[b, 0:1, :] = halo[b, 0:1, :] + tail_buf[...]
    for b in range(batch):
        pltpu.make_async_copy(
            halo.at[b], out_ref.at[b, pl.ds(seq_len, 1), :], tail_sem
        ).start()
    for b in range(batch):
        pltpu.make_async_copy(
            halo.at[b], out_ref.at[b, pl.ds(seq_len, 1), :], tail_sem
        ).wait()

    def drain(i, _):
        out_dma(i).wait()
        return ()
    jax.lax.fori_loop(steps - nbuf, steps, drain, (), unroll=False)


def kernel(x, cls_token, pos_table):
    batch, seq_len, d = x.shape
    s = _S
    nbuf = _NBUF
    kx = seq_len // s

    out = pl.pallas_call(
        functools.partial(_body, batch=batch, seq_len=seq_len, d=d, s=s,
                          nbuf=nbuf),
        in_specs=[
            pl.BlockSpec(memory_space=pltpu.MemorySpace.HBM),
            pl.BlockSpec((1, 1, d), lambda: (0, 0, 0)),
            pl.BlockSpec(memory_space=pltpu.MemorySpace.HBM),
        ],
        out_specs=pl.BlockSpec(memory_space=pltpu.MemorySpace.HBM),
        out_shape=jax.ShapeDtypeStruct((batch, seq_len + 1, d), x.dtype),
        scratch_shapes=[
            pltpu.VMEM((nbuf, s, d), x.dtype),
            pltpu.VMEM((nbuf, s, d), x.dtype),
            pltpu.VMEM((seq_len, d), x.dtype),
            pltpu.VMEM((batch, 1, d), x.dtype),
            pltpu.VMEM((1, d), x.dtype),
            pltpu.SemaphoreType.DMA((nbuf,)),
            pltpu.SemaphoreType.DMA((nbuf,)),
            pltpu.SemaphoreType.DMA((kx,)),
            pltpu.SemaphoreType.DMA,
        ],
    )(x, cls_token, pos_table)
    return out
